# baseline (speedup 1.0000x reference)
import functools

import jax
import jax.numpy as jnp
from jax import lax
from jax.experimental import pallas as pl
from jax.experimental.pallas import tpu as pltpu
from jax.experimental.pallas import tpu_sc as plsc

BATCH = 16384
D = 64
NC = 2
NS = 16
NW = NC * NS
BPW = BATCH // NW
NCHUNK = BPW // 128
NGRP = BPW // 16

_mesh = plsc.VectorSubcoreMesh(core_axis_name="c", subcore_axis_name="s")


@functools.partial(
    pl.kernel,
    mesh=_mesh,
    out_type=jax.ShapeDtypeStruct((BATCH,), jnp.float32),
    scratch_types=[
        pltpu.VMEM((NCHUNK, 128), jnp.int32),
        pltpu.VMEM((NCHUNK, 128), jnp.int32),
        pltpu.VMEM((BPW, D), jnp.float32),
        pltpu.VMEM((BPW, D), jnp.float32),
        pltpu.VMEM((BPW,), jnp.float32),
        pltpu.SemaphoreType.DMA,
    ],
    compiler_params=pltpu.CompilerParams(needs_layout_passes=False,
                                         use_tc_tiling_on_sc=False),
)
def _bpr_sc(user_hbm, item_hbm, uemb_hbm, iemb_hbm, out_hbm,
            uidx_v, iidx_v, urows_v, irows_v, out_v, sem):
    wid = lax.axis_index("s") * NC + lax.axis_index("c")
    base = wid * BPW

    pltpu.sync_copy(user_hbm.at[pl.ds(wid * NCHUNK, NCHUNK)], uidx_v)
    pltpu.sync_copy(item_hbm.at[pl.ds(wid * NCHUNK, NCHUNK)], iidx_v)

    copies = []
    for j in range(NCHUNK):
        copies.append(pltpu.async_copy(
            uemb_hbm.at[uidx_v.at[j]], urows_v.at[pl.ds(j * 128, 128)], sem))
        copies.append(pltpu.async_copy(
            iemb_hbm.at[iidx_v.at[j]], irows_v.at[pl.ds(j * 128, 128)], sem))
    for c in copies:
        c.wait()

    lane = lax.iota(jnp.int32, 16)

    def group(g, carry):
        row0 = g * 16
        acc = jnp.zeros((16,), jnp.float32)
        for r in range(16):
            row = row0 + r
            p = urows_v[row, pl.ds(0, 16)] * irows_v[row, pl.ds(0, 16)]
            for c in range(1, D // 16):
                p = p + (urows_v[row, pl.ds(c * 16, 16)]
                         * irows_v[row, pl.ds(c * 16, 16)])
            acc = jnp.where(lane == r, jnp.sum(p), acc)
        out_v[pl.ds(row0, 16)] = acc
        return carry

    lax.fori_loop(0, NGRP, group, 0)
    pltpu.sync_copy(out_v, out_hbm.at[pl.ds(base, BPW)])


def kernel(user, item, user_emb, item_emb):
    user2d = user.astype(jnp.int32).reshape(NW * NCHUNK, 128)
    item2d = item.astype(jnp.int32).reshape(NW * NCHUNK, 128)
    return _bpr_sc(user2d, item2d, user_emb, item_emb)


# speedup vs baseline: 2.0648x; 2.0648x over previous
import functools

import jax
import jax.numpy as jnp
from jax import lax
from jax.experimental import pallas as pl
from jax.experimental.pallas import tpu as pltpu
from jax.experimental.pallas import tpu_sc as plsc

BATCH = 16384
D = 64
NC = 2
NS = 16
NW = NC * NS
BPW = BATCH // NW
NBLK = BPW // 16
RING = 4
LOOK = 2

_mesh = plsc.VectorSubcoreMesh(core_axis_name="c", subcore_axis_name="s")


@functools.partial(
    pl.kernel,
    mesh=_mesh,
    out_type=jax.ShapeDtypeStruct((BATCH,), jnp.float32),
    scratch_types=(
        [
            pltpu.VMEM((BPW,), jnp.int32),
            pltpu.VMEM((BPW,), jnp.int32),
            pltpu.VMEM((RING, 8, D), jnp.float32),
            pltpu.VMEM((RING, D, 128), jnp.float32),
            pltpu.VMEM((BPW,), jnp.float32),
        ]
        + [pltpu.SemaphoreType.DMA] * (2 * RING)
    ),
    compiler_params=pltpu.CompilerParams(needs_layout_passes=False,
                                         disable_bounds_checks=True),
)
def _bpr_sc(user_hbm, item_hbm, uemb_hbm, itemT_hbm, out_hbm,
            uidx_v, iidx_v, uring_v, iring_v, out_v, *sems):
    usems, isems = sems[:RING], sems[RING:]
    wid = lax.axis_index("s") * NC + lax.axis_index("c")
    base = wid * BPW

    pltpu.sync_copy(user_hbm.at[pl.ds(base, BPW)], uidx_v)
    pltpu.sync_copy(item_hbm.at[pl.ds(base, BPW)], iidx_v)

    def fire(slot, uv, iv):
        uoff = pl.multiple_of((uv >> 3) * 8, 8)
        pltpu.async_copy(uemb_hbm.at[pl.ds(uoff, 8), :],
                         uring_v.at[slot], usems[slot])
        ioff = pl.multiple_of((iv >> 7) * 128, 128)
        pltpu.async_copy(itemT_hbm.at[:, pl.ds(ioff, 128)],
                         iring_v.at[slot], isems[slot])

    uvec0 = uidx_v[pl.ds(0, 16)]
    ivec0 = iidx_v[pl.ds(0, 16)]
    for j in range(LOOK):
        fire(j % RING, uvec0[j], ivec0[j])

    lane = lax.iota(jnp.int32, 16)
    d16 = [lane + 16 * cb for cb in range(D // 16)]

    def block(blk, carry):
        j0 = blk * 16
        uvec = uidx_v[pl.ds(j0, 16)]
        ivec = iidx_v[pl.ds(j0, 16)]
        nxt0 = jnp.minimum(j0 + 16, BPW - 16)
        uvec_n = uidx_v[pl.ds(nxt0, 16)]
        ivec_n = iidx_v[pl.ds(nxt0, 16)]
        acc = jnp.zeros((16,), jnp.float32)
        for r in range(16):
            j = j0 + r
            if r + LOOK < 16:
                uvf, ivf = uvec[r + LOOK], ivec[r + LOOK]
            else:
                uvf, ivf = uvec_n[r + LOOK - 16], ivec_n[r + LOOK - 16]
            slot_f = (r + LOOK) % RING

            @pl.when(j + LOOK < BPW)
            def _():
                fire(slot_f, uvf, ivf)

            slot = r % RING
            pltpu.make_async_copy(uemb_hbm.at[pl.ds(0, 8), :],
                                  uring_v.at[slot], usems[slot]).wait()
            pltpu.make_async_copy(itemT_hbm.at[:, pl.ds(0, 128)],
                                  iring_v.at[slot], isems[slot]).wait()
            urow = uvec[r] & 7
            l = jnp.broadcast_to(ivec[r] & 127, (16,))
            p = (uring_v[slot, urow, pl.ds(0, 16)]
                 * plsc.load_gather(iring_v.at[slot], [d16[0], l]))
            for cb in range(1, D // 16):
                p = p + (uring_v[slot, urow, pl.ds(cb * 16, 16)]
                         * plsc.load_gather(iring_v.at[slot], [d16[cb], l]))
            acc = jnp.where(lane == r, jnp.sum(p), acc)
        out_v[pl.ds(j0, 16)] = acc
        return carry

    lax.fori_loop(0, NBLK, block, 0)
    pltpu.sync_copy(out_v, out_hbm.at[pl.ds(base, BPW)])


def kernel(user, item, user_emb, item_emb):
    return _bpr_sc(user.astype(jnp.int32), item.astype(jnp.int32),
                   user_emb, item_emb.T)


# speedup vs baseline: 2.5373x; 1.2288x over previous
import functools

import jax
import jax.numpy as jnp
from jax import lax
from jax.experimental import pallas as pl
from jax.experimental.pallas import tpu as pltpu
from jax.experimental.pallas import tpu_sc as plsc

BATCH = 16384
D = 64
NC = 2
NS = 16
NW = NC * NS
BPW = BATCH // NW
NBLK = BPW // 16
RING = 8
LOOK = 4

_mesh = plsc.VectorSubcoreMesh(core_axis_name="c", subcore_axis_name="s")


@functools.partial(
    pl.kernel,
    mesh=_mesh,
    out_type=jax.ShapeDtypeStruct((BATCH,), jnp.float32),
    scratch_types=(
        [
            pltpu.VMEM((BPW,), jnp.int32),
            pltpu.VMEM((BPW,), jnp.int32),
            pltpu.VMEM((RING, 8, D), jnp.float32),
            pltpu.VMEM((RING, D, 128), jnp.float32),
            pltpu.VMEM((BPW,), jnp.float32),
        ]
        + [pltpu.SemaphoreType.DMA] * (2 * RING)
    ),
    compiler_params=pltpu.CompilerParams(needs_layout_passes=False,
                                         disable_bounds_checks=True),
)
def _bpr_sc(user_hbm, item_hbm, uemb_hbm, itemT_hbm, out_hbm,
            uidx_v, iidx_v, uring_v, iring_v, out_v, *sems):
    usems, isems = sems[:RING], sems[RING:]
    wid = lax.axis_index("s") * NC + lax.axis_index("c")
    base = wid * BPW

    pltpu.sync_copy(user_hbm.at[pl.ds(base, BPW)], uidx_v)
    pltpu.sync_copy(item_hbm.at[pl.ds(base, BPW)], iidx_v)

    def fire(slot, uv, iv):
        uoff = pl.multiple_of((uv >> 3) * 8, 8)
        pltpu.async_copy(uemb_hbm.at[pl.ds(uoff, 8), :],
                         uring_v.at[slot], usems[slot])
        ioff = pl.multiple_of((iv >> 7) * 128, 128)
        pltpu.async_copy(itemT_hbm.at[:, pl.ds(ioff, 128)],
                         iring_v.at[slot], isems[slot])

    uvec0 = uidx_v[pl.ds(0, 16)]
    ivec0 = iidx_v[pl.ds(0, 16)]
    for j in range(LOOK):
        fire(j % RING, uvec0[j], ivec0[j])

    lane = lax.iota(jnp.int32, 16)
    d16 = [lane + 16 * cb for cb in range(D // 16)]

    def block(blk, carry):
        j0 = blk * 16
        uvec = uidx_v[pl.ds(j0, 16)]
        ivec = iidx_v[pl.ds(j0, 16)]
        nxt0 = jnp.minimum(j0 + 16, BPW - 16)
        uvec_n = uidx_v[pl.ds(nxt0, 16)]
        ivec_n = iidx_v[pl.ds(nxt0, 16)]
        acc = jnp.zeros((16,), jnp.float32)
        for r in range(16):
            j = j0 + r
            if r + LOOK < 16:
                uvf, ivf = uvec[r + LOOK], ivec[r + LOOK]
            else:
                uvf, ivf = uvec_n[r + LOOK - 16], ivec_n[r + LOOK - 16]
            slot_f = (r + LOOK) % RING

            @pl.when(j + LOOK < BPW)
            def _():
                fire(slot_f, uvf, ivf)

            slot = r % RING
            pltpu.make_async_copy(uemb_hbm.at[pl.ds(0, 8), :],
                                  uring_v.at[slot], usems[slot]).wait()
            pltpu.make_async_copy(itemT_hbm.at[:, pl.ds(0, 128)],
                                  iring_v.at[slot], isems[slot]).wait()
            urow = uvec[r] & 7
            l = jnp.broadcast_to(ivec[r] & 127, (16,))
            p = (uring_v[slot, urow, pl.ds(0, 16)]
                 * plsc.load_gather(iring_v.at[slot], [d16[0], l]))
            for cb in range(1, D // 16):
                p = p + (uring_v[slot, urow, pl.ds(cb * 16, 16)]
                         * plsc.load_gather(iring_v.at[slot], [d16[cb], l]))
            acc = jnp.where(lane == r, jnp.sum(p), acc)
        out_v[pl.ds(j0, 16)] = acc
        return carry

    lax.fori_loop(0, NBLK, block, 0)
    pltpu.sync_copy(out_v, out_hbm.at[pl.ds(base, BPW)])


def kernel(user, item, user_emb, item_emb):
    return _bpr_sc(user.astype(jnp.int32), item.astype(jnp.int32),
                   user_emb, item_emb.T)
